# Initial kernel scaffold; baseline (speedup 1.0000x reference)
import functools

import jax
import jax.numpy as jnp
from jax import lax
from jax.experimental import pallas as pl
from jax.experimental.pallas import tpu as pltpu
from jax.experimental.pallas import tpu_sc as plsc

N = 10000
D = 128
DH = 64
NC = 2
NS = 16
ROWS_PER_TILE = N // NS
NPAD = N + 8
HROWS = 10016
HCHUNK = HROWS // NS
EBLK = 128


def _mesh():
    return plsc.VectorSubcoreMesh(core_axis_name="c", subcore_axis_name="s")


def _hist_body(n_blocks, dst_hbm, ones_hbm, zeros_hbm, out_hbm,
               hist_sp, ones_v, didx_v):
    c = lax.axis_index("c")
    s = lax.axis_index("s")
    pltpu.sync_copy(zeros_hbm, hist_sp.at[pl.ds(s * HCHUNK, HCHUNK)])
    pltpu.sync_copy(ones_hbm, ones_v)
    plsc.subcore_barrier()
    wid = c * NS + s
    base = wid * (n_blocks * EBLK)

    def blk(i, carry):
        pltpu.sync_copy(dst_hbm.at[pl.ds(base + i * EBLK, EBLK)], didx_v)
        pltpu.sync_copy(ones_v, hist_sp.at[didx_v], add=True)
        return carry

    lax.fori_loop(0, n_blocks, blk, 0)
    plsc.subcore_barrier()
    pltpu.sync_copy(hist_sp.at[pl.ds(s * HCHUNK, HCHUNK)],
                    out_hbm.at[c, pl.ds(s * HCHUNK, HCHUNK)])


def _hist(dst_pad, ones, zeros):
    n_blocks = dst_pad.shape[0] // (NC * NS * EBLK)
    body = functools.partial(_hist_body, n_blocks)
    return pl.kernel(
        body,
        out_type=jax.ShapeDtypeStruct((NC, HROWS, 16), jnp.float32),
        mesh=_mesh(),
        scratch_types=[
            pltpu.VMEM_SHARED((HROWS, 16), jnp.float32),
            pltpu.VMEM((EBLK, 16), jnp.float32),
            pltpu.VMEM((EBLK,), jnp.int32),
        ],
    )(dst_pad, ones, zeros)


def _hop_body(n_blocks, x_hbm, src_hbm, dst_hbm, zeros_hbm, out_hbm,
              g_sp, acc_sp, sidx_v, didx_v, rows_v):
    c = lax.axis_index("c")
    s = lax.axis_index("s")
    r0 = s * ROWS_PER_TILE
    pltpu.sync_copy(x_hbm.at[c, pl.ds(r0, ROWS_PER_TILE)],
                    g_sp.at[pl.ds(r0, ROWS_PER_TILE)])
    pltpu.sync_copy(zeros_hbm, acc_sp.at[pl.ds(r0, ROWS_PER_TILE)])

    @pl.when(s == NS - 1)
    def _():
        pltpu.sync_copy(zeros_hbm.at[pl.ds(0, NPAD - N)],
                        g_sp.at[pl.ds(N, NPAD - N)])
        pltpu.sync_copy(zeros_hbm.at[pl.ds(0, NPAD - N)],
                        acc_sp.at[pl.ds(N, NPAD - N)])

    plsc.subcore_barrier()
    base = s * (n_blocks * EBLK)

    def blk(i, carry):
        e0 = base + i * EBLK
        pltpu.sync_copy(src_hbm.at[pl.ds(e0, EBLK)], sidx_v)
        pltpu.sync_copy(dst_hbm.at[pl.ds(e0, EBLK)], didx_v)
        pltpu.sync_copy(g_sp.at[sidx_v], rows_v)
        pltpu.sync_copy(rows_v, acc_sp.at[didx_v], add=True)
        return carry

    lax.fori_loop(0, n_blocks, blk, 0)
    plsc.subcore_barrier()
    pltpu.sync_copy(acc_sp.at[pl.ds(r0, ROWS_PER_TILE)],
                    out_hbm.at[c, pl.ds(r0, ROWS_PER_TILE)])


def _hop(x, src_pad, dst_pad, zeros):
    n_blocks = src_pad.shape[0] // (NS * EBLK)
    body = functools.partial(_hop_body, n_blocks)
    return pl.kernel(
        body,
        out_type=jax.ShapeDtypeStruct((NC, N, DH), jnp.float32),
        mesh=_mesh(),
        scratch_types=[
            pltpu.VMEM_SHARED((NPAD, DH), jnp.float32),
            pltpu.VMEM_SHARED((NPAD, DH), jnp.float32),
            pltpu.VMEM((EBLK,), jnp.int32),
            pltpu.VMEM((EBLK,), jnp.int32),
            pltpu.VMEM((EBLK, DH), jnp.float32),
        ],
    )(x, src_pad, dst_pad, zeros)


def _deg_from_hist(hist):
    deg = hist[0, :N, 0] + hist[1, :N, 0]
    return jnp.maximum(deg, 1.0)


def _prescale_body(hist_ref, feat_ref, o_ref):
    norm = lax.rsqrt(_deg_from_hist(hist_ref[...]))
    f = feat_ref[...] * norm[:, None]
    o_ref[0] = f[:, :DH]
    o_ref[1] = f[:, DH:]


def _prescale(hist, feat):
    return pl.pallas_call(
        _prescale_body,
        out_shape=jax.ShapeDtypeStruct((NC, N, DH), jnp.float32),
    )(hist, feat)


def _midscale_body(hist_ref, y_ref, o_ref):
    inv = 1.0 / _deg_from_hist(hist_ref[...])
    o_ref[...] = y_ref[...] * inv[None, :, None]


def _midscale(hist, y):
    return pl.pallas_call(
        _midscale_body,
        out_shape=jax.ShapeDtypeStruct((NC, N, DH), jnp.float32),
    )(hist, y)


def _final_body(hist_ref, y_ref, w_ref, b_ref, o_ref):
    norm = lax.rsqrt(_deg_from_hist(hist_ref[...]))
    h = jnp.concatenate([y_ref[0], y_ref[1]], axis=1) * norm[:, None]
    mean = jnp.mean(h, axis=0)
    cen = h - mean[None, :]
    var = jnp.sum(cen * cen, axis=0) / (N - 1)
    xn = cen / jnp.sqrt(var)[None, :]
    out = lax.dot_general(xn, w_ref[...], (((1,), (1,)), ((), ())),
                          preferred_element_type=jnp.float32)
    o_ref[...] = out + b_ref[...][None, :]


def _final(hist, y, W, b):
    return pl.pallas_call(
        _final_body,
        out_shape=jax.ShapeDtypeStruct((N, D), jnp.float32),
    )(hist, y, W, b)


def kernel(feat, edge_index, W, b):
    E = edge_index.shape[1]
    quant = NC * NS * EBLK
    e_pad = ((E + quant - 1) // quant) * quant
    pad = jnp.full((e_pad - E,), N, dtype=jnp.int32)
    src = jnp.concatenate([edge_index[0].astype(jnp.int32), pad])
    dst = jnp.concatenate([edge_index[1].astype(jnp.int32), pad])

    ones = jnp.ones((EBLK, 16), jnp.float32)
    zeros_h = jnp.zeros((HCHUNK, 16), jnp.float32)
    zeros_c = jnp.zeros((ROWS_PER_TILE, DH), jnp.float32)

    hist = _hist(dst, ones, zeros_h)
    g0 = _prescale(hist, feat)
    y1 = _hop(g0, src, dst, zeros_c)
    g1 = _midscale(hist, y1)
    y2 = _hop(g1, src, dst, zeros_c)
    return _final(hist, y2, W, b)


# speedup vs baseline: 4.0727x; 4.0727x over previous
import functools

import jax
import jax.numpy as jnp
from jax import lax
from jax.experimental import pallas as pl
from jax.experimental.pallas import tpu as pltpu
from jax.experimental.pallas import tpu_sc as plsc

N = 10000
D = 128
DH = 64
NC = 2
NS = 16
NR = 10112
RPT = NR // NS
EBLK = 128


def _mesh():
    return plsc.VectorSubcoreMesh(core_axis_name="c", subcore_axis_name="s")


def _hist_body(n_blocks, dst_hbm, ones_hbm, zeros_hbm, out_hbm,
               hist_sp, ones_v, didx_v):
    c = lax.axis_index("c")
    s = lax.axis_index("s")
    pltpu.sync_copy(zeros_hbm, hist_sp.at[pl.ds(s * RPT, RPT)])
    pltpu.sync_copy(ones_hbm, ones_v)
    plsc.subcore_barrier()
    wid = c * NS + s
    base = wid * (n_blocks * EBLK)

    def blk(i, carry):
        pltpu.sync_copy(dst_hbm.at[pl.ds(base + i * EBLK, EBLK)], didx_v)
        pltpu.sync_copy(ones_v, hist_sp.at[didx_v], add=True)
        return carry

    lax.fori_loop(0, n_blocks, blk, 0)
    plsc.subcore_barrier()
    pltpu.sync_copy(hist_sp.at[pl.ds(s * RPT, RPT)],
                    out_hbm.at[c, pl.ds(s * RPT, RPT)])


def _hist(dst_pad, ones, zeros):
    n_blocks = dst_pad.shape[0] // (NC * NS * EBLK)
    body = functools.partial(_hist_body, n_blocks)
    return pl.kernel(
        body,
        out_type=jax.ShapeDtypeStruct((NC, NR, 16), jnp.float32),
        mesh=_mesh(),
        compiler_params=pltpu.CompilerParams(use_tc_tiling_on_sc=False),
        scratch_types=[
            pltpu.VMEM_SHARED((NR, 16), jnp.float32),
            pltpu.VMEM((EBLK, 16), jnp.float32),
            pltpu.VMEM((EBLK,), jnp.int32),
        ],
    )(dst_pad, ones, zeros)


def _hop_body(n_blocks, x_hbm, src_hbm, dst_hbm, zeros_hbm, out_hbm,
              g_sp, acc_sp, sidx_v, didx_v, rows_v):
    c = lax.axis_index("c")
    s = lax.axis_index("s")
    r0 = s * RPT
    pltpu.sync_copy(x_hbm.at[c, pl.ds(r0, RPT)], g_sp.at[pl.ds(r0, RPT)])
    pltpu.sync_copy(zeros_hbm, acc_sp.at[pl.ds(r0, RPT)])
    plsc.subcore_barrier()
    base = s * (n_blocks * EBLK)

    def blk(i, carry):
        e0 = base + i * EBLK
        pltpu.sync_copy(src_hbm.at[pl.ds(e0, EBLK)], sidx_v)
        pltpu.sync_copy(dst_hbm.at[pl.ds(e0, EBLK)], didx_v)
        pltpu.sync_copy(g_sp.at[sidx_v], rows_v)
        pltpu.sync_copy(rows_v, acc_sp.at[didx_v], add=True)
        return carry

    lax.fori_loop(0, n_blocks, blk, 0)
    plsc.subcore_barrier()
    pltpu.sync_copy(acc_sp.at[pl.ds(r0, RPT)], out_hbm.at[c, pl.ds(r0, RPT)])


def _hop(x, src_pad, dst_pad, zeros):
    n_blocks = src_pad.shape[0] // (NS * EBLK)
    body = functools.partial(_hop_body, n_blocks)
    return pl.kernel(
        body,
        out_type=jax.ShapeDtypeStruct((NC, NR, DH), jnp.float32),
        mesh=_mesh(),
        compiler_params=pltpu.CompilerParams(use_tc_tiling_on_sc=False),
        scratch_types=[
            pltpu.VMEM_SHARED((NR, DH), jnp.float32),
            pltpu.VMEM_SHARED((NR, DH), jnp.float32),
            pltpu.VMEM((EBLK,), jnp.int32),
            pltpu.VMEM((EBLK,), jnp.int32),
            pltpu.VMEM((EBLK, DH), jnp.float32),
        ],
    )(x, src_pad, dst_pad, zeros)


def _deg_from_hist(hist):
    deg = hist[0, :, 0] + hist[1, :, 0]
    return jnp.maximum(deg, 1.0)


def _prescale_body(hist_ref, feat_ref, o_ref):
    norm = lax.rsqrt(_deg_from_hist(hist_ref[...]))[:N]
    f = feat_ref[...] * norm[:, None]
    zpad = jnp.zeros((NR - N, DH), jnp.float32)
    o_ref[0] = jnp.concatenate([f[:, :DH], zpad], axis=0)
    o_ref[1] = jnp.concatenate([f[:, DH:], zpad], axis=0)


def _prescale(hist, feat):
    return pl.pallas_call(
        _prescale_body,
        out_shape=jax.ShapeDtypeStruct((NC, NR, DH), jnp.float32),
    )(hist, feat)


def _midscale_body(hist_ref, y_ref, o_ref):
    inv = 1.0 / _deg_from_hist(hist_ref[...])
    o_ref[...] = y_ref[...] * inv[None, :, None]


def _midscale(hist, y):
    return pl.pallas_call(
        _midscale_body,
        out_shape=jax.ShapeDtypeStruct((NC, NR, DH), jnp.float32),
    )(hist, y)


def _final_body(hist_ref, y_ref, w_ref, b_ref, o_ref):
    norm = lax.rsqrt(_deg_from_hist(hist_ref[...]))[:N]
    h = jnp.concatenate([y_ref[0, :N], y_ref[1, :N]], axis=1) * norm[:, None]
    mean = jnp.mean(h, axis=0)
    cen = h - mean[None, :]
    var = jnp.sum(cen * cen, axis=0) / (N - 1)
    xn = cen / jnp.sqrt(var)[None, :]
    out = lax.dot_general(xn, w_ref[...], (((1,), (1,)), ((), ())),
                          preferred_element_type=jnp.float32)
    o_ref[...] = out + b_ref[...][None, :]


def _final(hist, y, W, b):
    return pl.pallas_call(
        _final_body,
        out_shape=jax.ShapeDtypeStruct((N, D), jnp.float32),
    )(hist, y, W, b)


def kernel(feat, edge_index, W, b):
    E = edge_index.shape[1]
    quant = NC * NS * EBLK
    e_pad = ((E + quant - 1) // quant) * quant
    pad = jnp.full((e_pad - E,), N, dtype=jnp.int32)
    src = jnp.concatenate([edge_index[0].astype(jnp.int32), pad])
    dst = jnp.concatenate([edge_index[1].astype(jnp.int32), pad])

    ones = jnp.ones((EBLK, 16), jnp.float32)
    zeros_h = jnp.zeros((RPT, 16), jnp.float32)
    zeros_c = jnp.zeros((RPT, DH), jnp.float32)

    hist = _hist(dst, ones, zeros_h)
    g0 = _prescale(hist, feat)
    y1 = _hop(g0, src, dst, zeros_c)
    g1 = _midscale(hist, y1)
    y2 = _hop(g1, src, dst, zeros_c)
    return _final(hist, y2, W, b)


# speedup vs baseline: 7.3927x; 1.8152x over previous
import functools

import jax
import jax.numpy as jnp
from jax import lax
from jax.experimental import pallas as pl
from jax.experimental.pallas import tpu as pltpu
from jax.experimental.pallas import tpu_sc as plsc

N = 10000
D = 128
DH = 64
NC = 2
NS = 16
NR = 10112
RPT = NR // NS
EBLK = 128
N_MACRO = 2


def _mesh():
    return plsc.VectorSubcoreMesh(core_axis_name="c", subcore_axis_name="s")


def _hist_body(n_blocks, dst_hbm, ones_hbm, zeros_hbm, out_hbm,
               hist_sp, ones_v, didx, asem):
    c = lax.axis_index("c")
    s = lax.axis_index("s")
    pltpu.sync_copy(zeros_hbm, hist_sp.at[pl.ds(s * RPT, RPT)])
    pltpu.sync_copy(ones_hbm, ones_v)
    wid = c * NS + s
    pltpu.sync_copy(dst_hbm.at[pl.ds(wid * n_blocks, n_blocks)], didx)
    plsc.subcore_barrier()

    win = 8

    def blk(i, carry):
        pltpu.async_copy(ones_v, hist_sp.at[didx.at[i]], asem, add=True)

        @pl.when(i >= win)
        def _():
            pltpu.make_async_copy(ones_v, hist_sp.at[didx.at[i]], asem).wait()
        return carry

    lax.fori_loop(0, n_blocks, blk, 0)

    def drain(i, carry):
        pltpu.make_async_copy(ones_v, hist_sp.at[didx.at[0]], asem).wait()
        return carry

    lax.fori_loop(0, min(win, n_blocks), drain, 0)
    plsc.subcore_barrier()
    pltpu.sync_copy(hist_sp.at[pl.ds(s * RPT, RPT)],
                    out_hbm.at[c, pl.ds(s * RPT, RPT)])


def _hist(dst2d, ones, zeros):
    n_blocks = dst2d.shape[0] // (NC * NS)
    body = functools.partial(_hist_body, n_blocks)
    return pl.kernel(
        body,
        out_type=jax.ShapeDtypeStruct((NC, NR, 16), jnp.float32),
        mesh=_mesh(),
        compiler_params=pltpu.CompilerParams(use_tc_tiling_on_sc=False),
        scratch_types=[
            pltpu.VMEM_SHARED((NR, 16), jnp.float32),
            pltpu.VMEM((EBLK, 16), jnp.float32),
            pltpu.VMEM((n_blocks, EBLK), jnp.int32),
            pltpu.SemaphoreType.DMA,
        ],
    )(dst2d, ones, zeros)


def _hop_body(n_blocks, x_hbm, src_hbm, dst_hbm, zeros_hbm, out_hbm,
              g_sp, acc_sp, sidx, didx, rows, gsem, ssem):
    c = lax.axis_index("c")
    s = lax.axis_index("s")
    r0 = s * RPT
    pltpu.sync_copy(x_hbm.at[c, pl.ds(r0, RPT)], g_sp.at[pl.ds(r0, RPT)])
    pltpu.sync_copy(zeros_hbm, acc_sp.at[pl.ds(r0, RPT)])
    plsc.subcore_barrier()
    mchunk = n_blocks // N_MACRO

    def macro(m, mcarry):
        b0 = s * n_blocks + m * mchunk
        pltpu.sync_copy(src_hbm.at[pl.ds(b0, mchunk)], sidx)
        pltpu.sync_copy(dst_hbm.at[pl.ds(b0, mchunk)], didx)
        pltpu.async_copy(g_sp.at[sidx.at[0]], rows.at[0], gsem.at[0])

        def blk(i, carry):
            j = lax.rem(i, 2)
            jn = lax.rem(i + 1, 2)

            @pl.when(i + 1 < mchunk)
            def _():
                @pl.when(i >= 1)
                def _():
                    pltpu.make_async_copy(
                        rows.at[jn], acc_sp.at[didx.at[i]],
                        ssem.at[jn]).wait()
                pltpu.async_copy(g_sp.at[sidx.at[i + 1]], rows.at[jn],
                                 gsem.at[jn])

            pltpu.make_async_copy(g_sp.at[sidx.at[i]], rows.at[j],
                                  gsem.at[j]).wait()
            pltpu.async_copy(rows.at[j], acc_sp.at[didx.at[i]], ssem.at[j],
                             add=True)
            return carry

        lax.fori_loop(0, mchunk, blk, 0)
        j_last = (mchunk - 1) % 2
        pltpu.make_async_copy(rows.at[j_last], acc_sp.at[didx.at[0]],
                              ssem.at[j_last]).wait()
        pltpu.make_async_copy(rows.at[1 - j_last], acc_sp.at[didx.at[0]],
                              ssem.at[1 - j_last]).wait()
        return mcarry

    lax.fori_loop(0, N_MACRO, macro, 0)
    plsc.subcore_barrier()
    pltpu.sync_copy(acc_sp.at[pl.ds(r0, RPT)], out_hbm.at[c, pl.ds(r0, RPT)])


def _hop(x, src2d, dst2d, zeros):
    n_blocks = src2d.shape[0] // NS
    assert n_blocks % N_MACRO == 0
    mchunk = n_blocks // N_MACRO
    body = functools.partial(_hop_body, n_blocks)
    return pl.kernel(
        body,
        out_type=jax.ShapeDtypeStruct((NC, NR, DH), jnp.float32),
        mesh=_mesh(),
        compiler_params=pltpu.CompilerParams(use_tc_tiling_on_sc=False),
        scratch_types=[
            pltpu.VMEM_SHARED((NR, DH), jnp.float32),
            pltpu.VMEM_SHARED((NR, DH), jnp.float32),
            pltpu.VMEM((mchunk, EBLK), jnp.int32),
            pltpu.VMEM((mchunk, EBLK), jnp.int32),
            pltpu.VMEM((2, EBLK, DH), jnp.float32),
            pltpu.SemaphoreType.DMA((2,)),
            pltpu.SemaphoreType.DMA((2,)),
        ],
    )(x, src2d, dst2d, zeros)


def _deg_from_hist(hist):
    deg = hist[0, :, 0] + hist[1, :, 0]
    return jnp.maximum(deg, 1.0)


def _prescale_body(hist_ref, feat_ref, o_ref):
    norm = lax.rsqrt(_deg_from_hist(hist_ref[...]))[:N]
    f = feat_ref[...] * norm[:, None]
    zpad = jnp.zeros((NR - N, DH), jnp.float32)
    o_ref[0] = jnp.concatenate([f[:, :DH], zpad], axis=0)
    o_ref[1] = jnp.concatenate([f[:, DH:], zpad], axis=0)


def _prescale(hist, feat):
    return pl.pallas_call(
        _prescale_body,
        out_shape=jax.ShapeDtypeStruct((NC, NR, DH), jnp.float32),
    )(hist, feat)


def _midscale_body(hist_ref, y_ref, o_ref):
    inv = 1.0 / _deg_from_hist(hist_ref[...])
    o_ref[...] = y_ref[...] * inv[None, :, None]


def _midscale(hist, y):
    return pl.pallas_call(
        _midscale_body,
        out_shape=jax.ShapeDtypeStruct((NC, NR, DH), jnp.float32),
    )(hist, y)


def _final_body(hist_ref, y_ref, w_ref, b_ref, o_ref):
    norm = lax.rsqrt(_deg_from_hist(hist_ref[...]))[:N]
    h = jnp.concatenate([y_ref[0, :N], y_ref[1, :N]], axis=1) * norm[:, None]
    mean = jnp.mean(h, axis=0)
    cen = h - mean[None, :]
    var = jnp.sum(cen * cen, axis=0) / (N - 1)
    xn = cen / jnp.sqrt(var)[None, :]
    out = lax.dot_general(xn, w_ref[...], (((1,), (1,)), ((), ())),
                          preferred_element_type=jnp.float32)
    o_ref[...] = out + b_ref[...][None, :]


def _final(hist, y, W, b):
    return pl.pallas_call(
        _final_body,
        out_shape=jax.ShapeDtypeStruct((N, D), jnp.float32),
    )(hist, y, W, b)


def kernel(feat, edge_index, W, b):
    E = edge_index.shape[1]
    quant = NC * NS * EBLK
    e_pad = ((E + quant - 1) // quant) * quant
    pad = jnp.full((e_pad - E,), N, dtype=jnp.int32)
    src = jnp.concatenate([edge_index[0].astype(jnp.int32), pad]).reshape(-1, EBLK)
    dst = jnp.concatenate([edge_index[1].astype(jnp.int32), pad]).reshape(-1, EBLK)

    ones = jnp.ones((EBLK, 16), jnp.float32)
    zeros_h = jnp.zeros((RPT, 16), jnp.float32)
    zeros_c = jnp.zeros((RPT, DH), jnp.float32)

    hist = _hist(dst, ones, zeros_h)
    g0 = _prescale(hist, feat)
    y1 = _hop(g0, src, dst, zeros_c)
    g1 = _midscale(hist, y1)
    y2 = _hop(g1, src, dst, zeros_c)
    return _final(hist, y2, W, b)
